# baseline (speedup 1.0000x reference)
import jax
import jax.numpy as jnp
import numpy as np
from jax import lax
from jax.experimental import pallas as pl
from jax.experimental.pallas import tpu as pltpu
from jax.experimental.pallas import tpu_sc as plsc

N_NODES = 10000
N_GENES = 128
N_EDGES = 320000
GC = 2 * N_GENES

NC = 2
NS = 16
NW = NC * NS
LANES = 16
GW = GC // 2
WPT = GW // NW
EC = 8000
NCHE = N_EDGES // EC
EGRP = EC // LANES

ROW_BLK = 2000
NBLK = N_NODES // ROW_BLK


def _moments_body(yh_ref, yt_ref, ev_ref, mu_ref, w_ref, acc_ref, wacc_ref):
    i = pl.program_id(0)

    @pl.when(i == 0)
    def _():
        acc_ref[...] = jnp.zeros_like(acc_ref)
        wacc_ref[...] = jnp.zeros_like(wacc_ref)

    acc_ref[:, :N_GENES] += jnp.sum(yh_ref[...], axis=0, keepdims=True)
    acc_ref[:, N_GENES:] += jnp.sum(yt_ref[...], axis=0, keepdims=True)
    wacc_ref[...] += jnp.sum(ev_ref[...])[None, None]

    @pl.when(i == NBLK - 1)
    def _():
        mu_ref[...] = acc_ref[...] / N_NODES
        w_ref[...] = wacc_ref[...]


def _moments(y_hat, y_true, ev2d):
    return pl.pallas_call(
        _moments_body,
        grid=(NBLK,),
        in_specs=[
            pl.BlockSpec((ROW_BLK, N_GENES), lambda i: (i, 0)),
            pl.BlockSpec((ROW_BLK, N_GENES), lambda i: (i, 0)),
            pl.BlockSpec((ROW_BLK, N_EDGES // N_NODES), lambda i: (i, 0)),
        ],
        out_specs=[
            pl.BlockSpec((1, GC), lambda i: (0, 0)),
            pl.BlockSpec((1, 1), lambda i: (0, 0)),
        ],
        out_shape=[
            jax.ShapeDtypeStruct((1, GC), jnp.float32),
            jax.ShapeDtypeStruct((1, 1), jnp.float32),
        ],
        scratch_shapes=[
            pltpu.VMEM((1, GC), jnp.float32),
            pltpu.VMEM((1, 1), jnp.float32),
        ],
    )(y_hat, y_true, ev2d)


def _center_body(yh_ref, yt_ref, mu_ref, c_ref, den_ref, dacc_ref):
    i = pl.program_id(0)

    @pl.when(i == 0)
    def _():
        dacc_ref[...] = jnp.zeros_like(dacc_ref)

    ch = yh_ref[...] - mu_ref[0:1, :N_GENES]
    ct = yt_ref[...] - mu_ref[0:1, N_GENES:]
    c_ref[:, :N_GENES] = ch.astype(jnp.bfloat16)
    c_ref[:, N_GENES:] = ct.astype(jnp.bfloat16)
    dacc_ref[:, :N_GENES] += jnp.sum(ch * ch, axis=0, keepdims=True)
    dacc_ref[:, N_GENES:] += jnp.sum(ct * ct, axis=0, keepdims=True)

    @pl.when(i == NBLK - 1)
    def _():
        den_ref[...] = dacc_ref[...]


def _center(y_hat, y_true, mu):
    return pl.pallas_call(
        _center_body,
        grid=(NBLK,),
        in_specs=[
            pl.BlockSpec((ROW_BLK, N_GENES), lambda i: (i, 0)),
            pl.BlockSpec((ROW_BLK, N_GENES), lambda i: (i, 0)),
            pl.BlockSpec((1, GC), lambda i: (0, 0)),
        ],
        out_specs=[
            pl.BlockSpec((ROW_BLK, GC), lambda i: (i, 0)),
            pl.BlockSpec((1, GC), lambda i: (0, 0)),
        ],
        out_shape=[
            jax.ShapeDtypeStruct((N_NODES, GC), jnp.bfloat16),
            jax.ShapeDtypeStruct((1, GC), jnp.float32),
        ],
        scratch_shapes=[pltpu.VMEM((1, GC), jnp.float32)],
    )(y_hat, y_true, mu)


def _edge_body(t_hbm, src_hbm, dst_hbm, w_hbm, out_hbm,
               tbl, sb0, sb1, db0, db1, wb0, wb1, accbuf, sem0, sem1):
    wid = lax.axis_index("s") * NC + lax.axis_index("c")

    pltpu.sync_copy(t_hbm.at[pl.ds(WPT * wid, WPT)], tbl)

    bufs = ((sb0, db0, wb0, sem0), (sb1, db1, wb1, sem1))

    def issue(c, b):
        sb, db, wb, sem = bufs[b]
        off = pl.multiple_of(c * EC, 8)
        pltpu.async_copy(src_hbm.at[pl.ds(off, EC)], sb, sem)
        pltpu.async_copy(dst_hbm.at[pl.ds(off, EC)], db, sem)
        pltpu.async_copy(w_hbm.at[pl.ds(off, EC)], wb, sem)

    def wait(b):
        sb, db, wb, sem = bufs[b]
        pltpu.make_async_copy(src_hbm.at[pl.ds(0, EC)], sb, sem).wait()
        pltpu.make_async_copy(dst_hbm.at[pl.ds(0, EC)], db, sem).wait()
        pltpu.make_async_copy(w_hbm.at[pl.ds(0, EC)], wb, sem).wait()

    def compute(b, accs):
        sb, db, wb, _ = bufs[b]

        def one_grp(g, accs):
            si = sb[pl.ds(g * LANES, LANES)]
            di = db[pl.ds(g * LANES, LANES)]
            wpk = plsc.bitcast(wb[pl.ds(g * LANES, LANES)], jnp.bfloat16)
            new = []
            for gw in range(WPT):
                row = jnp.full((LANES,), gw, jnp.int32)
                s = plsc.load_gather(tbl, [row, si])
                d = plsc.load_gather(tbl, [row, di])
                p = plsc.bitcast(s, jnp.bfloat16) * plsc.bitcast(d, jnp.bfloat16)
                pa, pb = plsc.unpack(
                    p * wpk, format=plsc.PackFormat.INTERLEAVED)
                new.append(accs[2 * gw] + pa)
                new.append(accs[2 * gw + 1] + pb)
            return tuple(new)

        def grp_body(h, accs):
            accs = one_grp(2 * h, accs)
            return one_grp(2 * h + 1, accs)

        return lax.fori_loop(0, EGRP // 2, grp_body, accs)

    issue(0, 0)

    def pair_body(k, accs):
        c0 = 2 * k
        wait(0)
        issue(c0 + 1, 1)
        accs = compute(0, accs)
        wait(1)

        @pl.when(c0 + 2 < NCHE)
        def _():
            issue(c0 + 2, 0)

        return compute(1, accs)

    accs = tuple(jnp.zeros((LANES,), jnp.float32) for _ in range(2 * WPT))
    accs = lax.fori_loop(0, NCHE // 2, pair_body, accs)

    for j in range(2 * WPT):
        accbuf[j, :] = accs[j]
    pltpu.sync_copy(accbuf, out_hbm.at[pl.ds(2 * WPT * wid, 2 * WPT)])


def _edge_partials(c32t, src, dst, edge_vals):
    mesh = plsc.VectorSubcoreMesh(
        core_axis_name="c", subcore_axis_name="s",
        num_cores=NC, num_subcores=NS)
    return pl.kernel(
        _edge_body,
        out_type=jax.ShapeDtypeStruct((GC, LANES), jnp.float32),
        mesh=mesh,
        compiler_params=pltpu.CompilerParams(needs_layout_passes=False),
        scratch_types=[
            pltpu.VMEM((WPT, N_NODES), jnp.int32),
            pltpu.VMEM((EC,), jnp.int32),
            pltpu.VMEM((EC,), jnp.int32),
            pltpu.VMEM((EC,), jnp.int32),
            pltpu.VMEM((EC,), jnp.int32),
            pltpu.VMEM((EC,), jnp.int32),
            pltpu.VMEM((EC,), jnp.int32),
            pltpu.VMEM((2 * WPT, LANES), jnp.float32),
            pltpu.SemaphoreType.DMA,
            pltpu.SemaphoreType.DMA,
        ],
    )(c32t, src, dst, edge_vals)


def _final_body(p_ref, den_ref, w_ref, out_ref):
    num = jnp.sum(p_ref[...], axis=1, keepdims=True)
    den = den_ref[...]
    den = den + jnp.where(den == 0.0, 1e-6, 0.0)
    stats = (N_NODES / w_ref[0, 0]) * num / den
    diff = stats[:N_GENES, 0:1] - stats[N_GENES:, 0:1]
    out_ref[...] = jnp.mean(diff * diff)[None, None]


def _final(partials, den, w):
    return pl.pallas_call(
        _final_body,
        out_shape=jax.ShapeDtypeStruct((1, 1), jnp.float32),
    )(partials, den, w)


def kernel(Y_hat, Y_true, edge_index, edge_vals):
    ev2d = edge_vals.reshape(N_NODES, N_EDGES // N_NODES)
    mu, w = _moments(Y_hat, Y_true, ev2d)
    c_cat, den = _center(Y_hat, Y_true, mu)
    c32t = lax.bitcast_convert_type(
        c_cat.reshape(N_NODES, GW, 2), jnp.int32).T
    wbf = edge_vals.astype(jnp.bfloat16)
    w_pk = lax.bitcast_convert_type(
        jnp.stack([wbf, wbf], axis=-1), jnp.int32)
    partials = _edge_partials(c32t, edge_index[0], edge_index[1], w_pk)
    loss = _final(partials, den.reshape(GC, 1), w)
    return loss[0, 0]


# speedup vs baseline: 1.0153x; 1.0153x over previous
import jax
import jax.numpy as jnp
import numpy as np
from jax import lax
from jax.experimental import pallas as pl
from jax.experimental.pallas import tpu as pltpu
from jax.experimental.pallas import tpu_sc as plsc

N_NODES = 10000
N_GENES = 128
N_EDGES = 320000
GC = 2 * N_GENES

NC = 2
NS = 16
NW = NC * NS
LANES = 16
GW = GC // 2
WPT = GW // NW
EC = 8000
NCHE = N_EDGES // EC
EGRP = EC // LANES

ROW_BLK = 2000
NBLK = N_NODES // ROW_BLK


def _prep_body(yh_ref, yt_ref, ev_ref, c_ref, den_ref, w_ref,
               acc_ref, wacc_ref, mu_ref, dacc_ref):
    p = pl.program_id(0)
    i = pl.program_id(1)

    @pl.when((p == 0) & (i == 0))
    def _():
        acc_ref[...] = jnp.zeros_like(acc_ref)
        wacc_ref[...] = jnp.zeros_like(wacc_ref)
        dacc_ref[...] = jnp.zeros_like(dacc_ref)

    @pl.when(p == 0)
    def _():
        acc_ref[:, :N_GENES] += jnp.sum(yh_ref[...], axis=0, keepdims=True)
        acc_ref[:, N_GENES:] += jnp.sum(yt_ref[...], axis=0, keepdims=True)
        wacc_ref[...] += jnp.sum(ev_ref[...])[None, None]

    @pl.when(p == 1)
    def _():
        @pl.when(i == 0)
        def _():
            mu_ref[...] = acc_ref[...] / N_NODES

        ch = yh_ref[...] - mu_ref[0:1, :N_GENES]
        ct = yt_ref[...] - mu_ref[0:1, N_GENES:]
        c_ref[:, :N_GENES] = ch.astype(jnp.bfloat16)
        c_ref[:, N_GENES:] = ct.astype(jnp.bfloat16)
        dacc_ref[:, :N_GENES] += jnp.sum(ch * ch, axis=0, keepdims=True)
        dacc_ref[:, N_GENES:] += jnp.sum(ct * ct, axis=0, keepdims=True)

        @pl.when(i == NBLK - 1)
        def _():
            den_ref[...] = dacc_ref[...]
            w_ref[...] = wacc_ref[...]


def _prep(y_hat, y_true, ev2d):
    return pl.pallas_call(
        _prep_body,
        grid=(2, NBLK),
        in_specs=[
            pl.BlockSpec((ROW_BLK, N_GENES), lambda p, i: (i, 0)),
            pl.BlockSpec((ROW_BLK, N_GENES), lambda p, i: (i, 0)),
            pl.BlockSpec((ROW_BLK, N_EDGES // N_NODES), lambda p, i: (i, 0)),
        ],
        out_specs=[
            pl.BlockSpec((ROW_BLK, GC), lambda p, i: (i, 0)),
            pl.BlockSpec((1, GC), lambda p, i: (0, 0)),
            pl.BlockSpec((1, 1), lambda p, i: (0, 0)),
        ],
        out_shape=[
            jax.ShapeDtypeStruct((N_NODES, GC), jnp.bfloat16),
            jax.ShapeDtypeStruct((1, GC), jnp.float32),
            jax.ShapeDtypeStruct((1, 1), jnp.float32),
        ],
        scratch_shapes=[
            pltpu.VMEM((1, GC), jnp.float32),
            pltpu.VMEM((1, 1), jnp.float32),
            pltpu.VMEM((1, GC), jnp.float32),
            pltpu.VMEM((1, GC), jnp.float32),
        ],
    )(y_hat, y_true, ev2d)


def _edge_body(t_hbm, src_hbm, dst_hbm, w_hbm, out_hbm,
               tbl, sb0, sb1, db0, db1, wb0, wb1, accbuf, sem0, sem1):
    wid = lax.axis_index("s") * NC + lax.axis_index("c")

    pltpu.sync_copy(t_hbm.at[pl.ds(WPT * wid, WPT)], tbl)

    bufs = ((sb0, db0, wb0, sem0), (sb1, db1, wb1, sem1))

    def issue(c, b):
        sb, db, wb, sem = bufs[b]
        off = pl.multiple_of(c * EC, 8)
        pltpu.async_copy(src_hbm.at[pl.ds(off, EC)], sb, sem)
        pltpu.async_copy(dst_hbm.at[pl.ds(off, EC)], db, sem)
        pltpu.async_copy(w_hbm.at[pl.ds(off, EC)], wb, sem)

    def wait(b):
        sb, db, wb, sem = bufs[b]
        pltpu.make_async_copy(src_hbm.at[pl.ds(0, EC)], sb, sem).wait()
        pltpu.make_async_copy(dst_hbm.at[pl.ds(0, EC)], db, sem).wait()
        pltpu.make_async_copy(w_hbm.at[pl.ds(0, EC)], wb, sem).wait()

    def compute(b, accs):
        sb, db, wb, _ = bufs[b]

        def one_grp(g, accs):
            si = sb[pl.ds(g * LANES, LANES)]
            di = db[pl.ds(g * LANES, LANES)]
            wpk = plsc.bitcast(wb[pl.ds(g * LANES, LANES)], jnp.bfloat16)
            new = []
            for gw in range(WPT):
                row = jnp.full((LANES,), gw, jnp.int32)
                s = plsc.load_gather(tbl, [row, si])
                d = plsc.load_gather(tbl, [row, di])
                p = plsc.bitcast(s, jnp.bfloat16) * plsc.bitcast(d, jnp.bfloat16)
                pa, pb = plsc.unpack(
                    p * wpk, format=plsc.PackFormat.INTERLEAVED)
                new.append(accs[2 * gw] + pa)
                new.append(accs[2 * gw + 1] + pb)
            return tuple(new)

        return lax.fori_loop(0, EGRP, one_grp, accs)

    issue(0, 0)

    def pair_body(k, accs):
        c0 = 2 * k
        wait(0)
        issue(c0 + 1, 1)
        accs = compute(0, accs)
        wait(1)

        @pl.when(c0 + 2 < NCHE)
        def _():
            issue(c0 + 2, 0)

        return compute(1, accs)

    accs = tuple(jnp.zeros((LANES,), jnp.float32) for _ in range(2 * WPT))
    accs = lax.fori_loop(0, NCHE // 2, pair_body, accs)

    for j in range(2 * WPT):
        accbuf[j, :] = accs[j]
    pltpu.sync_copy(accbuf, out_hbm.at[pl.ds(2 * WPT * wid, 2 * WPT)])


def _edge_partials(c32t, src, dst, edge_vals):
    mesh = plsc.VectorSubcoreMesh(
        core_axis_name="c", subcore_axis_name="s",
        num_cores=NC, num_subcores=NS)
    return pl.kernel(
        _edge_body,
        out_type=jax.ShapeDtypeStruct((GC, LANES), jnp.float32),
        mesh=mesh,
        compiler_params=pltpu.CompilerParams(needs_layout_passes=False),
        scratch_types=[
            pltpu.VMEM((WPT, N_NODES), jnp.int32),
            pltpu.VMEM((EC,), jnp.int32),
            pltpu.VMEM((EC,), jnp.int32),
            pltpu.VMEM((EC,), jnp.int32),
            pltpu.VMEM((EC,), jnp.int32),
            pltpu.VMEM((EC,), jnp.int32),
            pltpu.VMEM((EC,), jnp.int32),
            pltpu.VMEM((2 * WPT, LANES), jnp.float32),
            pltpu.SemaphoreType.DMA,
            pltpu.SemaphoreType.DMA,
        ],
    )(c32t, src, dst, edge_vals)


def _final_body(p_ref, den_ref, w_ref, out_ref):
    num = jnp.sum(p_ref[...], axis=1, keepdims=True)
    den = den_ref[...]
    den = den + jnp.where(den == 0.0, 1e-6, 0.0)
    stats = (N_NODES / w_ref[0, 0]) * num / den
    diff = stats[:N_GENES, 0:1] - stats[N_GENES:, 0:1]
    out_ref[...] = jnp.mean(diff * diff)[None, None]


def _final(partials, den, w):
    return pl.pallas_call(
        _final_body,
        out_shape=jax.ShapeDtypeStruct((1, 1), jnp.float32),
    )(partials, den, w)


def kernel(Y_hat, Y_true, edge_index, edge_vals):
    ev2d = edge_vals.reshape(N_NODES, N_EDGES // N_NODES)
    c_cat, den, w = _prep(Y_hat, Y_true, ev2d)
    c32t = lax.bitcast_convert_type(
        c_cat.reshape(N_NODES, GW, 2), jnp.int32).T
    wbf = edge_vals.astype(jnp.bfloat16)
    w_pk = lax.bitcast_convert_type(
        jnp.stack([wbf, wbf], axis=-1), jnp.int32)
    partials = _edge_partials(c32t, edge_index[0], edge_index[1], w_pk)
    loss = _final(partials, den.reshape(GC, 1), w)
    return loss[0, 0]


# speedup vs baseline: 1.0365x; 1.0209x over previous
import jax
import jax.numpy as jnp
import numpy as np
from jax import lax
from jax.experimental import pallas as pl
from jax.experimental.pallas import tpu as pltpu
from jax.experimental.pallas import tpu_sc as plsc

N_NODES = 10000
N_GENES = 128
N_EDGES = 320000
GC = 2 * N_GENES

NC = 2
NS = 16
NW = NC * NS
LANES = 16
GW = GC // 2
WPT = GW // NW
EC = 8000
NCHE = N_EDGES // EC
EGRP = EC // LANES

ROW_BLK = 2000
NBLK = N_NODES // ROW_BLK


def _moments_body(yh_ref, yt_ref, ev_ref, mu_ref, w_ref, acc_ref, wacc_ref):
    i = pl.program_id(0)

    @pl.when(i == 0)
    def _():
        acc_ref[...] = jnp.zeros_like(acc_ref)
        wacc_ref[...] = jnp.zeros_like(wacc_ref)

    acc_ref[:, :N_GENES] += jnp.sum(yh_ref[...], axis=0, keepdims=True)
    acc_ref[:, N_GENES:] += jnp.sum(yt_ref[...], axis=0, keepdims=True)
    wacc_ref[...] += jnp.sum(ev_ref[...])[None, None]

    @pl.when(i == NBLK - 1)
    def _():
        mu_ref[...] = acc_ref[...] / N_NODES
        w_ref[...] = wacc_ref[...]


def _moments(y_hat, y_true, ev2d):
    return pl.pallas_call(
        _moments_body,
        grid=(NBLK,),
        in_specs=[
            pl.BlockSpec((ROW_BLK, N_GENES), lambda i: (i, 0)),
            pl.BlockSpec((ROW_BLK, N_GENES), lambda i: (i, 0)),
            pl.BlockSpec((ROW_BLK, N_EDGES // N_NODES), lambda i: (i, 0)),
        ],
        out_specs=[
            pl.BlockSpec((1, GC), lambda i: (0, 0)),
            pl.BlockSpec((1, 1), lambda i: (0, 0)),
        ],
        out_shape=[
            jax.ShapeDtypeStruct((1, GC), jnp.float32),
            jax.ShapeDtypeStruct((1, 1), jnp.float32),
        ],
        scratch_shapes=[
            pltpu.VMEM((1, GC), jnp.float32),
            pltpu.VMEM((1, 1), jnp.float32),
        ],
    )(y_hat, y_true, ev2d)


def _center_body(yh_ref, yt_ref, mu_ref, c_ref, den_ref, dacc_ref):
    i = pl.program_id(0)

    @pl.when(i == 0)
    def _():
        dacc_ref[...] = jnp.zeros_like(dacc_ref)

    ch = yh_ref[...] - mu_ref[0:1, :N_GENES]
    ct = yt_ref[...] - mu_ref[0:1, N_GENES:]
    c_ref[:, :N_GENES] = ch.astype(jnp.bfloat16)
    c_ref[:, N_GENES:] = ct.astype(jnp.bfloat16)
    dacc_ref[:, :N_GENES] += jnp.sum(ch * ch, axis=0, keepdims=True)
    dacc_ref[:, N_GENES:] += jnp.sum(ct * ct, axis=0, keepdims=True)

    @pl.when(i == NBLK - 1)
    def _():
        den_ref[...] = dacc_ref[...]


def _center(y_hat, y_true, mu):
    return pl.pallas_call(
        _center_body,
        grid=(NBLK,),
        in_specs=[
            pl.BlockSpec((ROW_BLK, N_GENES), lambda i: (i, 0)),
            pl.BlockSpec((ROW_BLK, N_GENES), lambda i: (i, 0)),
            pl.BlockSpec((1, GC), lambda i: (0, 0)),
        ],
        out_specs=[
            pl.BlockSpec((ROW_BLK, GC), lambda i: (i, 0)),
            pl.BlockSpec((1, GC), lambda i: (0, 0)),
        ],
        out_shape=[
            jax.ShapeDtypeStruct((N_NODES, GC), jnp.bfloat16),
            jax.ShapeDtypeStruct((1, GC), jnp.float32),
        ],
        scratch_shapes=[pltpu.VMEM((1, GC), jnp.float32)],
    )(y_hat, y_true, mu)


def _edge_body(t_hbm, src_hbm, dst_hbm, w_hbm, out_hbm,
               tbl, sb0, sb1, db0, db1, wb0, wb1, accbuf, sem0, sem1):
    wid = lax.axis_index("s") * NC + lax.axis_index("c")

    pltpu.sync_copy(t_hbm.at[pl.ds(WPT * wid, WPT)], tbl)

    bufs = ((sb0, db0, wb0, sem0), (sb1, db1, wb1, sem1))

    def issue(c, b):
        sb, db, wb, sem = bufs[b]
        off = pl.multiple_of(c * EC, 8)
        pltpu.async_copy(src_hbm.at[pl.ds(off, EC)], sb, sem)
        pltpu.async_copy(dst_hbm.at[pl.ds(off, EC)], db, sem)
        pltpu.async_copy(w_hbm.at[pl.ds(off, EC)], wb, sem)

    def wait(b):
        sb, db, wb, sem = bufs[b]
        pltpu.make_async_copy(src_hbm.at[pl.ds(0, EC)], sb, sem).wait()
        pltpu.make_async_copy(dst_hbm.at[pl.ds(0, EC)], db, sem).wait()
        pltpu.make_async_copy(w_hbm.at[pl.ds(0, EC)], wb, sem).wait()

    def compute(b, accs):
        sb, db, wb, _ = bufs[b]

        himask = jnp.full((LANES,), -65536, jnp.int32)

        def one_grp(g, accs):
            si = sb[pl.ds(g * LANES, LANES)]
            di = db[pl.ds(g * LANES, LANES)]
            wv = wb[pl.ds(g * LANES, LANES)]
            new = []
            for gw in range(WPT):
                row = jnp.full((LANES,), gw, jnp.int32)
                s = plsc.load_gather(tbl, [row, si])
                d = plsc.load_gather(tbl, [row, di])
                p = plsc.bitcast(
                    plsc.bitcast(s, jnp.bfloat16) *
                    plsc.bitcast(d, jnp.bfloat16), jnp.int32)
                plo = plsc.bitcast(p << 16, jnp.float32)
                phi = plsc.bitcast(p & himask, jnp.float32)
                new.append(accs[2 * gw] + plo * wv)
                new.append(accs[2 * gw + 1] + phi * wv)
            return tuple(new)

        return lax.fori_loop(0, EGRP, one_grp, accs)

    issue(0, 0)

    def pair_body(k, accs):
        c0 = 2 * k
        wait(0)
        issue(c0 + 1, 1)
        accs = compute(0, accs)
        wait(1)

        @pl.when(c0 + 2 < NCHE)
        def _():
            issue(c0 + 2, 0)

        return compute(1, accs)

    accs = tuple(jnp.zeros((LANES,), jnp.float32) for _ in range(2 * WPT))
    accs = lax.fori_loop(0, NCHE // 2, pair_body, accs)

    for j in range(2 * WPT):
        accbuf[j, :] = accs[j]
    pltpu.sync_copy(accbuf, out_hbm.at[pl.ds(2 * WPT * wid, 2 * WPT)])


def _edge_partials(c32t, src, dst, edge_vals):
    mesh = plsc.VectorSubcoreMesh(
        core_axis_name="c", subcore_axis_name="s",
        num_cores=NC, num_subcores=NS)
    return pl.kernel(
        _edge_body,
        out_type=jax.ShapeDtypeStruct((GC, LANES), jnp.float32),
        mesh=mesh,
        compiler_params=pltpu.CompilerParams(needs_layout_passes=False),
        scratch_types=[
            pltpu.VMEM((WPT, N_NODES), jnp.int32),
            pltpu.VMEM((EC,), jnp.int32),
            pltpu.VMEM((EC,), jnp.int32),
            pltpu.VMEM((EC,), jnp.int32),
            pltpu.VMEM((EC,), jnp.int32),
            pltpu.VMEM((EC,), jnp.float32),
            pltpu.VMEM((EC,), jnp.float32),
            pltpu.VMEM((2 * WPT, LANES), jnp.float32),
            pltpu.SemaphoreType.DMA,
            pltpu.SemaphoreType.DMA,
        ],
    )(c32t, src, dst, edge_vals)


def _final_body(p_ref, den_ref, w_ref, out_ref):
    num = jnp.sum(p_ref[...], axis=1, keepdims=True)
    den = den_ref[...]
    den = den + jnp.where(den == 0.0, 1e-6, 0.0)
    stats = (N_NODES / w_ref[0, 0]) * num / den
    diff = stats[:N_GENES, 0:1] - stats[N_GENES:, 0:1]
    out_ref[...] = jnp.mean(diff * diff)[None, None]


def _final(partials, den, w):
    return pl.pallas_call(
        _final_body,
        out_shape=jax.ShapeDtypeStruct((1, 1), jnp.float32),
    )(partials, den, w)


def kernel(Y_hat, Y_true, edge_index, edge_vals):
    ev2d = edge_vals.reshape(N_NODES, N_EDGES // N_NODES)
    mu, w = _moments(Y_hat, Y_true, ev2d)
    c_cat, den = _center(Y_hat, Y_true, mu)
    c32t = lax.bitcast_convert_type(
        c_cat.reshape(N_NODES, GW, 2), jnp.int32).T
    partials = _edge_partials(c32t, edge_index[0], edge_index[1], edge_vals)
    loss = _final(partials, den.reshape(GC, 1), w)
    return loss[0, 0]


# speedup vs baseline: 1.0621x; 1.0247x over previous
import jax
import jax.numpy as jnp
import numpy as np
from jax import lax
from jax.experimental import pallas as pl
from jax.experimental.pallas import tpu as pltpu
from jax.experimental.pallas import tpu_sc as plsc

N_NODES = 10000
N_GENES = 128
N_EDGES = 320000
GC = 2 * N_GENES

NC = 2
NS = 16
NW = NC * NS
LANES = 16
PER_W = N_EDGES // NW
CHUNK = 80
NCHUNK = PER_W // CHUNK
NGRP = GC // LANES
GW = GC // 2

ROW_BLK = 2000
NBLK = N_NODES // ROW_BLK


def _moments_body(yh_ref, yt_ref, ev_ref, mu_ref, w_ref, acc_ref, wacc_ref):
    i = pl.program_id(0)

    @pl.when(i == 0)
    def _():
        acc_ref[...] = jnp.zeros_like(acc_ref)
        wacc_ref[...] = jnp.zeros_like(wacc_ref)

    acc_ref[:, :N_GENES] += jnp.sum(yh_ref[...], axis=0, keepdims=True)
    acc_ref[:, N_GENES:] += jnp.sum(yt_ref[...], axis=0, keepdims=True)
    wacc_ref[...] += jnp.sum(ev_ref[...])[None, None]

    @pl.when(i == NBLK - 1)
    def _():
        mu_ref[...] = acc_ref[...] / N_NODES
        w_ref[...] = wacc_ref[...]


def _moments(y_hat, y_true, ev2d):
    return pl.pallas_call(
        _moments_body,
        grid=(NBLK,),
        in_specs=[
            pl.BlockSpec((ROW_BLK, N_GENES), lambda i: (i, 0)),
            pl.BlockSpec((ROW_BLK, N_GENES), lambda i: (i, 0)),
            pl.BlockSpec((ROW_BLK, N_EDGES // N_NODES), lambda i: (i, 0)),
        ],
        out_specs=[
            pl.BlockSpec((1, GC), lambda i: (0, 0)),
            pl.BlockSpec((1, 1), lambda i: (0, 0)),
        ],
        out_shape=[
            jax.ShapeDtypeStruct((1, GC), jnp.float32),
            jax.ShapeDtypeStruct((1, 1), jnp.float32),
        ],
        scratch_shapes=[
            pltpu.VMEM((1, GC), jnp.float32),
            pltpu.VMEM((1, 1), jnp.float32),
        ],
    )(y_hat, y_true, ev2d)


def _center_body(yh_ref, yt_ref, mu_ref, c_ref, den_ref, dacc_ref):
    i = pl.program_id(0)

    @pl.when(i == 0)
    def _():
        dacc_ref[...] = jnp.zeros_like(dacc_ref)

    ch = yh_ref[...] - mu_ref[0:1, :N_GENES]
    ct = yt_ref[...] - mu_ref[0:1, N_GENES:]
    c_ref[:, :N_GENES] = ch.astype(jnp.bfloat16)
    c_ref[:, N_GENES:] = ct.astype(jnp.bfloat16)
    dacc_ref[:, :N_GENES] += jnp.sum(ch * ch, axis=0, keepdims=True)
    dacc_ref[:, N_GENES:] += jnp.sum(ct * ct, axis=0, keepdims=True)

    @pl.when(i == NBLK - 1)
    def _():
        den_ref[...] = dacc_ref[...]


def _center(y_hat, y_true, mu):
    return pl.pallas_call(
        _center_body,
        grid=(NBLK,),
        in_specs=[
            pl.BlockSpec((ROW_BLK, N_GENES), lambda i: (i, 0)),
            pl.BlockSpec((ROW_BLK, N_GENES), lambda i: (i, 0)),
            pl.BlockSpec((1, GC), lambda i: (0, 0)),
        ],
        out_specs=[
            pl.BlockSpec((ROW_BLK, GC), lambda i: (i, 0)),
            pl.BlockSpec((1, GC), lambda i: (0, 0)),
        ],
        out_shape=[
            jax.ShapeDtypeStruct((N_NODES, GC), jnp.bfloat16),
            jax.ShapeDtypeStruct((1, GC), jnp.float32),
        ],
        scratch_shapes=[pltpu.VMEM((1, GC), jnp.float32)],
    )(y_hat, y_true, mu)


def _edge_body(c_hbm, src_hbm, dst_hbm, w_hbm, out_hbm,
               srcv, dstv, wall, rs0, rs1, rd0, rd1, accbuf,
               sem0, sem1):
    wid = lax.axis_index("s") * NC + lax.axis_index("c")
    base0 = pl.multiple_of(wid * PER_W, 8)

    pltpu.sync_copy(src_hbm.at[pl.ds(base0, PER_W)], srcv)
    pltpu.sync_copy(dst_hbm.at[pl.ds(base0, PER_W)], dstv)
    pltpu.sync_copy(w_hbm.at[pl.ds(base0, PER_W)], wall)

    bufs = ((rs0, rd0, sem0), (rs1, rd1, sem1))

    def issue(c, b):
        rs, rd, sem = bufs[b]
        off = pl.multiple_of(c * CHUNK, 8)
        pltpu.async_copy(c_hbm.at[srcv.at[pl.ds(off, CHUNK)]], rs, sem)
        pltpu.async_copy(c_hbm.at[dstv.at[pl.ds(off, CHUNK)]], rd, sem)

    def wait(b):
        rs, rd, sem = bufs[b]
        pltpu.make_async_copy(c_hbm.at[srcv.at[pl.ds(0, CHUNK)]], rs, sem).wait()
        pltpu.make_async_copy(c_hbm.at[dstv.at[pl.ds(0, CHUNK)]], rd, sem).wait()

    def compute(c, b, accs):
        rs, rd, _ = bufs[b]

        def one_edge(e, accs):
            wv = plsc.load_gather(
                wall, [jnp.zeros((LANES,), jnp.int32) + (c * CHUNK + e)])
            wpk = plsc.pack(wv, wv, format=plsc.PackFormat.INTERLEAVED)
            new = []
            for j in range(NGRP // 2):
                sv = plsc.bitcast(rs[e, pl.ds(j * LANES, LANES)], jnp.bfloat16)
                dv = plsc.bitcast(rd[e, pl.ds(j * LANES, LANES)], jnp.bfloat16)
                pa, pb = plsc.unpack(
                    sv * dv * wpk, format=plsc.PackFormat.INTERLEAVED)
                new.append(accs[2 * j] + pa)
                new.append(accs[2 * j + 1] + pb)
            return tuple(new)

        def edge_body(h, accs):
            accs = one_edge(2 * h, accs)
            return one_edge(2 * h + 1, accs)

        return lax.fori_loop(0, CHUNK // 2, edge_body, accs)

    issue(0, 0)

    def pair_body(k, accs):
        c0 = 2 * k
        wait(0)
        issue(c0 + 1, 1)
        accs = compute(c0, 0, accs)
        wait(1)
        issue(c0 + 2, 0)
        return compute(c0 + 1, 1, accs)

    accs = tuple(jnp.zeros((LANES,), jnp.float32) for _ in range(NGRP))
    accs = lax.fori_loop(0, NCHUNK // 2, pair_body, accs)
    wait(0)
    accs = compute(NCHUNK - 1, 0, accs)

    for j in range(NGRP):
        accbuf[pl.ds(j * LANES, LANES)] = accs[j]
    pltpu.sync_copy(accbuf, out_hbm.at[wid])


def _edge_partials(c32, src, dst, edge_vals):
    mesh = plsc.VectorSubcoreMesh(
        core_axis_name="c", subcore_axis_name="s",
        num_cores=NC, num_subcores=NS)
    return pl.kernel(
        _edge_body,
        out_type=jax.ShapeDtypeStruct((NW, GC), jnp.float32),
        mesh=mesh,
        compiler_params=pltpu.CompilerParams(needs_layout_passes=False),
        scratch_types=[
            pltpu.VMEM((PER_W,), jnp.int32),
            pltpu.VMEM((PER_W,), jnp.int32),
            pltpu.VMEM((PER_W,), jnp.float32),
            pltpu.VMEM((CHUNK, GW), jnp.int32),
            pltpu.VMEM((CHUNK, GW), jnp.int32),
            pltpu.VMEM((CHUNK, GW), jnp.int32),
            pltpu.VMEM((CHUNK, GW), jnp.int32),
            pltpu.VMEM((GC,), jnp.float32),
            pltpu.SemaphoreType.DMA,
            pltpu.SemaphoreType.DMA,
        ],
    )(c32, src, dst, edge_vals)


def _final_body(p_ref, den_ref, w_ref, out_ref):
    num = jnp.sum(p_ref[...], axis=0, keepdims=True)
    den = den_ref[...]
    den = den + jnp.where(den == 0.0, 1e-6, 0.0)
    stats = (N_NODES / w_ref[0, 0]) * num / den
    diff = stats[0:1, :N_GENES] - stats[0:1, N_GENES:]
    out_ref[...] = jnp.mean(diff * diff)[None, None]


def _final(partials, den, w):
    return pl.pallas_call(
        _final_body,
        out_shape=jax.ShapeDtypeStruct((1, 1), jnp.float32),
    )(partials, den, w)


_POS = np.array(
    [32 * (g // 32) + (g % 32) // 2 + 16 * (g % 2) for g in range(GC)],
    dtype=np.int32)


def kernel(Y_hat, Y_true, edge_index, edge_vals):
    ev2d = edge_vals.reshape(N_NODES, N_EDGES // N_NODES)
    mu, w = _moments(Y_hat, Y_true, ev2d)
    c_cat, den = _center(Y_hat, Y_true, mu)
    c32 = lax.bitcast_convert_type(
        c_cat.reshape(N_NODES, GW, 2), jnp.int32)
    partials = _edge_partials(c32, edge_index[0], edge_index[1], edge_vals)
    loss = _final(partials[:, _POS], den, w)
    return loss[0, 0]
